# Initial kernel scaffold; baseline (speedup 1.0000x reference)
import jax
import jax.numpy as jnp
from jax.experimental import pallas as pl


def kernel(x, h_state, weight_ih, weight_hh, bias_ih, bias_hh, out_weight, out_bias):
    raise NotImplementedError("write your pallas kernel here")


# speedup vs baseline: 1.7894x; 1.7894x over previous
import functools

import jax
import jax.numpy as jnp
from jax import lax
from jax.experimental import pallas as pl
from jax.experimental.pallas import tpu as pltpu

_TIME_CHUNK = 512
_UNROLL = 8


def _rnn_kernel(x_ref, g0_ref, waug_ref, wih_ref, b_ref, wout_ref, bout_ref,
                y_ref, gfin_ref, g_carry, *, t_total, mask_tail, unroll):
    tc, bb = x_ref.shape
    hid = waug_ref.shape[1]
    half = bb // 2

    @pl.when(pl.program_id(1) == 0)
    def _():
        g_carry[...] = g0_ref[...]

    w_aug = waug_ref[...].astype(jnp.bfloat16)
    wih_h = jnp.broadcast_to(wih_ref[...], (hid, half))
    bias_h = jnp.broadcast_to(b_ref[...], (hid, half))
    bout_b = jnp.broadcast_to(bout_ref[...], (1, bb))

    t0 = pl.program_id(1) * tc

    def step(t, carry):
        g_a, g_b = carry
        pf_a = jnp.dot(w_aug, g_a.astype(jnp.bfloat16),
                       preferred_element_type=jnp.float32)
        pf_b = jnp.dot(w_aug, g_b.astype(jnp.bfloat16),
                       preferred_element_type=jnp.float32)
        y_prev = jnp.concatenate(
            [pf_a[hid:hid + 1], pf_b[hid:hid + 1]], axis=1) + bout_b
        tw = jnp.maximum(t - 1, 0)
        y_ref[pl.ds(tw, 1), :] = y_prev
        x_row = x_ref[pl.ds(t, 1), :]
        ga_new = jnp.tanh(pf_a[:hid] + wih_h * x_row[:, :half] + bias_h)
        gb_new = jnp.tanh(pf_b[:hid] + wih_h * x_row[:, half:] + bias_h)
        if mask_tail:
            keep = t0 + t < t_total
            ga_new = jnp.where(keep, ga_new, g_a)
            gb_new = jnp.where(keep, gb_new, g_b)
        return ga_new, gb_new

    g_a0 = g_carry[:, :half]
    g_b0 = g_carry[:, half:]
    g_a, g_b = lax.fori_loop(0, tc, step, (g_a0, g_b0), unroll=unroll)

    wout_h = jnp.broadcast_to(wout_ref[...], (hid, half))
    y_last = jnp.concatenate(
        [jnp.sum(g_a * wout_h, axis=0, keepdims=True),
         jnp.sum(g_b * wout_h, axis=0, keepdims=True)], axis=1) + bout_b
    y_ref[pl.ds(tc - 1, 1), :] = y_last

    g_fin = jnp.concatenate([g_a, g_b], axis=1)
    g_carry[...] = g_fin
    gfin_ref[...] = g_fin


def _rnn_forward(x_btf, h_state, weight_ih, weight_hh, bias_ih, bias_hh,
                 out_weight, out_bias):
    B, T, I = x_btf.shape
    H = weight_hh.shape[0]
    assert I == 1 and B % 4 == 0

    h0 = h_state[0].astype(jnp.float32)
    x_tb = jnp.transpose(x_btf[:, :, 0]).astype(jnp.float32)
    g0 = jnp.transpose(h0)

    w_hh = weight_hh.astype(jnp.float32)
    w_out_row = out_weight.reshape(1, H).astype(jnp.float32)
    w_aug = jnp.concatenate(
        [w_hh, w_out_row, jnp.zeros((7, H), jnp.float32)], axis=0)
    w_ih = weight_ih.reshape(H, 1).astype(jnp.float32)
    bias = (bias_ih + bias_hh).reshape(H, 1).astype(jnp.float32)
    w_out = out_weight.reshape(H, 1).astype(jnp.float32)
    b_out = out_bias.reshape(1, 1).astype(jnp.float32)

    if T <= _TIME_CHUNK:
        tc, t_pad = T, T
    else:
        tc = max(8, (_TIME_CHUNK // 8) * 8)
        t_pad = pl.cdiv(T, tc) * tc
    if t_pad != T:
        x_tb = jnp.pad(x_tb, ((0, t_pad - T), (0, 0)))
    n_chunks = t_pad // tc
    bb = B // 2

    kernel_fn = functools.partial(
        _rnn_kernel, t_total=T, mask_tail=(t_pad != T),
        unroll=min(tc, _UNROLL))

    y_tb, g_fin = pl.pallas_call(
        kernel_fn,
        grid=(2, n_chunks),
        in_specs=[
            pl.BlockSpec((tc, bb), lambda i, c: (c, i)),
            pl.BlockSpec((H, bb), lambda i, c: (0, i)),
            pl.BlockSpec((H + 8, H), lambda i, c: (0, 0)),
            pl.BlockSpec((H, 1), lambda i, c: (0, 0)),
            pl.BlockSpec((H, 1), lambda i, c: (0, 0)),
            pl.BlockSpec((H, 1), lambda i, c: (0, 0)),
            pl.BlockSpec((1, 1), lambda i, c: (0, 0)),
        ],
        out_specs=[
            pl.BlockSpec((tc, bb), lambda i, c: (c, i)),
            pl.BlockSpec((H, bb), lambda i, c: (0, i)),
        ],
        out_shape=[
            jax.ShapeDtypeStruct((t_pad, B), jnp.float32),
            jax.ShapeDtypeStruct((H, B), jnp.float32),
        ],
        scratch_shapes=[pltpu.VMEM((H, bb), jnp.float32)],
        compiler_params=pltpu.CompilerParams(
            dimension_semantics=("parallel", "arbitrary")),
    )(x_tb, g0, w_aug, w_ih, bias, w_out, b_out)

    y_btf = jnp.transpose(y_tb[:T])[:, :, None]
    return y_btf, jnp.transpose(g_fin)[None]


def kernel(x, h_state, weight_ih, weight_hh, bias_ih, bias_hh,
           out_weight, out_bias):
    return _rnn_forward(x, h_state, weight_ih, weight_hh, bias_ih, bias_hh,
                        out_weight, out_bias)


# speedup vs baseline: 2.4777x; 1.3847x over previous
import functools

import jax
import jax.numpy as jnp
import numpy as np
from jax import lax
from jax.experimental import pallas as pl
from jax.experimental.pallas import tpu as pltpu
from jax.sharding import Mesh, PartitionSpec as P

_TIME_CHUNK = 512
_UNROLL = 8


def _rnn_kernel(x_ref, g0_ref, waug_ref, wih_ref, b_ref, wout_ref, bout_ref,
                y_ref, gfin_ref, g_carry, *, t_total, mask_tail, unroll):
    tc, bb = x_ref.shape
    hid = waug_ref.shape[1]
    half = bb // 2

    @pl.when(pl.program_id(0) == 0)
    def _():
        g_carry[...] = g0_ref[...]

    w_aug = waug_ref[...].astype(jnp.bfloat16)
    wih_h = jnp.broadcast_to(wih_ref[...], (hid, half))
    bias_h = jnp.broadcast_to(b_ref[...], (hid, half))
    bout_b = jnp.broadcast_to(bout_ref[...], (1, bb))

    t0 = pl.program_id(0) * tc

    def step(t, carry):
        g_a, g_b = carry
        pf_a = jnp.dot(w_aug, g_a.astype(jnp.bfloat16),
                       preferred_element_type=jnp.float32)
        pf_b = jnp.dot(w_aug, g_b.astype(jnp.bfloat16),
                       preferred_element_type=jnp.float32)
        y_prev = jnp.concatenate(
            [pf_a[hid:hid + 1], pf_b[hid:hid + 1]], axis=1) + bout_b
        tw = jnp.maximum(t - 1, 0)
        y_ref[pl.ds(tw, 1), :] = y_prev
        x_row = x_ref[pl.ds(t, 1), :]
        ga_new = jnp.tanh(pf_a[:hid] + wih_h * x_row[:, :half] + bias_h)
        gb_new = jnp.tanh(pf_b[:hid] + wih_h * x_row[:, half:] + bias_h)
        if mask_tail:
            keep = t0 + t < t_total
            ga_new = jnp.where(keep, ga_new, g_a)
            gb_new = jnp.where(keep, gb_new, g_b)
        return ga_new, gb_new

    g_a0 = g_carry[:, :half]
    g_b0 = g_carry[:, half:]
    g_a, g_b = lax.fori_loop(0, tc, step, (g_a0, g_b0), unroll=unroll)

    wout_h = jnp.broadcast_to(wout_ref[...], (hid, half))
    y_last = jnp.concatenate(
        [jnp.sum(g_a * wout_h, axis=0, keepdims=True),
         jnp.sum(g_b * wout_h, axis=0, keepdims=True)], axis=1) + bout_b
    y_ref[pl.ds(tc - 1, 1), :] = y_last

    g_fin = jnp.concatenate([g_a, g_b], axis=1)
    g_carry[...] = g_fin
    gfin_ref[...] = g_fin


def _rnn_forward(x_btf, h_state, weight_ih, weight_hh, bias_ih, bias_hh,
                 out_weight, out_bias):
    B, T, I = x_btf.shape
    H = weight_hh.shape[0]
    assert I == 1 and B % 2 == 0

    h0 = h_state[0].astype(jnp.float32)
    x_tb = jnp.transpose(x_btf[:, :, 0]).astype(jnp.float32)
    g0 = jnp.transpose(h0)

    w_hh = weight_hh.astype(jnp.float32)
    w_out_row = out_weight.reshape(1, H).astype(jnp.float32)
    w_aug = jnp.concatenate(
        [w_hh, w_out_row, jnp.zeros((7, H), jnp.float32)], axis=0)
    w_ih = weight_ih.reshape(H, 1).astype(jnp.float32)
    bias = (bias_ih + bias_hh).reshape(H, 1).astype(jnp.float32)
    w_out = out_weight.reshape(H, 1).astype(jnp.float32)
    b_out = out_bias.reshape(1, 1).astype(jnp.float32)

    if T <= _TIME_CHUNK:
        tc, t_pad = T, T
    else:
        tc = max(8, (_TIME_CHUNK // 8) * 8)
        t_pad = pl.cdiv(T, tc) * tc
    if t_pad != T:
        x_tb = jnp.pad(x_tb, ((0, t_pad - T), (0, 0)))
    n_chunks = t_pad // tc

    kernel_fn = functools.partial(
        _rnn_kernel, t_total=T, mask_tail=(t_pad != T),
        unroll=min(tc, _UNROLL))

    y_tb, g_fin = pl.pallas_call(
        kernel_fn,
        grid=(n_chunks,),
        in_specs=[
            pl.BlockSpec((tc, B), lambda c: (c, 0)),
            pl.BlockSpec((H, B), lambda c: (0, 0)),
            pl.BlockSpec((H + 8, H), lambda c: (0, 0)),
            pl.BlockSpec((H, 1), lambda c: (0, 0)),
            pl.BlockSpec((H, 1), lambda c: (0, 0)),
            pl.BlockSpec((H, 1), lambda c: (0, 0)),
            pl.BlockSpec((1, 1), lambda c: (0, 0)),
        ],
        out_specs=[
            pl.BlockSpec((tc, B), lambda c: (c, 0)),
            pl.BlockSpec((H, B), lambda c: (0, 0)),
        ],
        out_shape=[
            jax.ShapeDtypeStruct((t_pad, B), jnp.float32),
            jax.ShapeDtypeStruct((H, B), jnp.float32),
        ],
        scratch_shapes=[pltpu.VMEM((H, B), jnp.float32)],
        compiler_params=pltpu.CompilerParams(
            dimension_semantics=("arbitrary",)),
    )(x_tb, g0, w_aug, w_ih, bias, w_out, b_out)

    y_btf = jnp.transpose(y_tb[:T])[:, :, None]
    return y_btf, jnp.transpose(g_fin)[None]


def kernel(x, h_state, weight_ih, weight_hh, bias_ih, bias_hh,
           out_weight, out_bias):
    args = (x, h_state, weight_ih, weight_hh, bias_ih, bias_hh,
            out_weight, out_bias)
    devs = jax.devices()
    if len(devs) < 2 or x.shape[0] % 2 != 0:
        return _rnn_forward(*args)
    mesh = Mesh(np.array(devs[:2]), ("d",))
    fwd = jax.shard_map(
        _rnn_forward, mesh=mesh,
        in_specs=(P("d"), P(None, "d"), P(), P(), P(), P(), P(), P()),
        out_specs=(P("d"), P(None, "d")),
        check_vma=False)
    return fwd(*args)


# speedup vs baseline: 2.5746x; 1.0391x over previous
import functools

import jax
import jax.numpy as jnp
import numpy as np
from jax import lax
from jax.experimental import pallas as pl
from jax.experimental.pallas import tpu as pltpu
from jax.sharding import Mesh, PartitionSpec as P

_TIME_CHUNK = 512
_UNROLL = 16


def _rnn_kernel(x_ref, h0_ref, waug_ref, wih_ref, b_ref, wout_ref, bout_ref,
                y_ref, hfin_ref, g_carry, xt_scr, y_scr,
                *, t_total, mask_tail, unroll):
    bb, tc = x_ref.shape
    hid = waug_ref.shape[1]
    half = bb // 2

    @pl.when(pl.program_id(0) == 0)
    def _():
        g_carry[...] = jnp.transpose(h0_ref[...])

    xt_scr[...] = jnp.transpose(x_ref[...])

    w_aug = waug_ref[...].astype(jnp.bfloat16)
    wih_h = jnp.broadcast_to(wih_ref[...], (hid, half))
    bias_h = jnp.broadcast_to(b_ref[...], (hid, half))
    bout_b = jnp.broadcast_to(bout_ref[...], (1, bb))

    t0 = pl.program_id(0) * tc

    def step(t, carry):
        g_a, g_b = carry
        pf_a = jnp.dot(w_aug, g_a.astype(jnp.bfloat16),
                       preferred_element_type=jnp.float32)
        pf_b = jnp.dot(w_aug, g_b.astype(jnp.bfloat16),
                       preferred_element_type=jnp.float32)
        y_prev = jnp.concatenate(
            [pf_a[hid:hid + 1], pf_b[hid:hid + 1]], axis=1) + bout_b
        tw = jnp.maximum(t - 1, 0)
        y_scr[pl.ds(tw, 1), :] = y_prev
        x_row = xt_scr[pl.ds(t, 1), :]
        ga_new = jnp.tanh(pf_a[:hid] + wih_h * x_row[:, :half] + bias_h)
        gb_new = jnp.tanh(pf_b[:hid] + wih_h * x_row[:, half:] + bias_h)
        if mask_tail:
            keep = t0 + t < t_total
            ga_new = jnp.where(keep, ga_new, g_a)
            gb_new = jnp.where(keep, gb_new, g_b)
        return ga_new, gb_new

    g_a0 = g_carry[:, :half]
    g_b0 = g_carry[:, half:]
    g_a, g_b = lax.fori_loop(0, tc, step, (g_a0, g_b0), unroll=unroll)

    wout_h = jnp.broadcast_to(wout_ref[...], (hid, half))
    y_last = jnp.concatenate(
        [jnp.sum(g_a * wout_h, axis=0, keepdims=True),
         jnp.sum(g_b * wout_h, axis=0, keepdims=True)], axis=1) + bout_b
    y_scr[pl.ds(tc - 1, 1), :] = y_last

    y_ref[...] = jnp.transpose(y_scr[...])

    g_fin = jnp.concatenate([g_a, g_b], axis=1)
    g_carry[...] = g_fin
    hfin_ref[...] = jnp.transpose(g_fin)


def _rnn_forward(x_btf, h_state, weight_ih, weight_hh, bias_ih, bias_hh,
                 out_weight, out_bias):
    B, T, I = x_btf.shape
    H = weight_hh.shape[0]
    assert I == 1 and B % 2 == 0

    h0 = h_state[0].astype(jnp.float32)
    x_bt = x_btf[:, :, 0].astype(jnp.float32)

    w_hh = weight_hh.astype(jnp.float32)
    w_out_row = out_weight.reshape(1, H).astype(jnp.float32)
    w_aug = jnp.concatenate(
        [w_hh, w_out_row, jnp.zeros((7, H), jnp.float32)], axis=0)
    w_ih = weight_ih.reshape(H, 1).astype(jnp.float32)
    bias = (bias_ih + bias_hh).reshape(H, 1).astype(jnp.float32)
    w_out = out_weight.reshape(H, 1).astype(jnp.float32)
    b_out = out_bias.reshape(1, 1).astype(jnp.float32)

    if T <= _TIME_CHUNK:
        tc, t_pad = T, T
    else:
        tc = max(8, (_TIME_CHUNK // 8) * 8)
        t_pad = pl.cdiv(T, tc) * tc
    if t_pad != T:
        x_bt = jnp.pad(x_bt, ((0, 0), (0, t_pad - T)))
    n_chunks = t_pad // tc

    kernel_fn = functools.partial(
        _rnn_kernel, t_total=T, mask_tail=(t_pad != T),
        unroll=min(tc, _UNROLL))

    y_bt, h_fin = pl.pallas_call(
        kernel_fn,
        grid=(n_chunks,),
        in_specs=[
            pl.BlockSpec((B, tc), lambda c: (0, c)),
            pl.BlockSpec((B, H), lambda c: (0, 0)),
            pl.BlockSpec((H + 8, H), lambda c: (0, 0)),
            pl.BlockSpec((H, 1), lambda c: (0, 0)),
            pl.BlockSpec((H, 1), lambda c: (0, 0)),
            pl.BlockSpec((H, 1), lambda c: (0, 0)),
            pl.BlockSpec((1, 1), lambda c: (0, 0)),
        ],
        out_specs=[
            pl.BlockSpec((B, tc), lambda c: (0, c)),
            pl.BlockSpec((B, H), lambda c: (0, 0)),
        ],
        out_shape=[
            jax.ShapeDtypeStruct((B, t_pad), jnp.float32),
            jax.ShapeDtypeStruct((B, H), jnp.float32),
        ],
        scratch_shapes=[
            pltpu.VMEM((H, B), jnp.float32),
            pltpu.VMEM((tc, B), jnp.float32),
            pltpu.VMEM((tc, B), jnp.float32),
        ],
        compiler_params=pltpu.CompilerParams(
            dimension_semantics=("arbitrary",)),
    )(x_bt, h0, w_aug, w_ih, bias, w_out, b_out)

    return y_bt[:, :T, None], h_fin[None]


def kernel(x, h_state, weight_ih, weight_hh, bias_ih, bias_hh,
           out_weight, out_bias):
    args = (x, h_state, weight_ih, weight_hh, bias_ih, bias_hh,
            out_weight, out_bias)
    devs = jax.devices()
    if len(devs) < 2 or x.shape[0] % 2 != 0:
        return _rnn_forward(*args)
    mesh = Mesh(np.array(devs[:2]), ("d",))
    fwd = jax.shard_map(
        _rnn_forward, mesh=mesh,
        in_specs=(P("d"), P(None, "d"), P(), P(), P(), P(), P(), P()),
        out_specs=(P("d"), P(None, "d")),
        check_vma=False)
    return fwd(*args)
